# baseline (speedup 1.0000x reference)
import jax
import jax.numpy as jnp
from jax import lax
from jax.experimental import pallas as pl
from jax.experimental.pallas import tpu as pltpu
from jax.experimental.pallas import tpu_sc as plsc

VOCAB = 100000
EMBED_DIM = 128
BATCH = 4096
HIST_LEN = 200

NC = 2
NS = 16
NW = NC * NS
NB = BATCH // NW
LPAD = 208
NV = LPAD // 16
NG = EMBED_DIM // 32
WBITS = 13
WROWS = 1 << WBITS
NWIN = 13
WLAST = VOCAB - (NWIN - 1) * WROWS
WSLICE = WROWS // NS
WLSLICE = WLAST // NS
GCAP = 104


def _sc_kernel(x_hbm, table_hbm, out_hbm, xbuf, tmp, hbuf, gbufa, gbufb,
               obuf, shared, smc, sema, semb, semst):
    wid = lax.axis_index("s") * NC + lax.axis_index("c")
    sid = lax.axis_index("s")
    base = wid * NB

    iota16 = lax.iota(jnp.int32, 16)
    ones16 = jnp.ones((16,), jnp.int32)
    zeros16 = jnp.zeros((16,), jnp.int32)
    himask = jnp.full((16,), jnp.int32(-65536))
    wmask = jnp.full((16,), jnp.int32(WROWS - 1))

    def stage_start(w):
        half = (w & 1) * WROWS

        @pl.when(w < NWIN - 1)
        def _():
            pltpu.make_async_copy(
                table_hbm.at[pl.ds(w * WROWS + sid * WSLICE, WSLICE)],
                shared.at[pl.ds(half + sid * WSLICE, WSLICE)],
                semst).start()

        @pl.when(w == NWIN - 1)
        def _():
            pltpu.make_async_copy(
                table_hbm.at[pl.ds((NWIN - 1) * WROWS + sid * WLSLICE,
                                   WLSLICE)],
                shared.at[pl.ds(half + sid * WLSLICE, WLSLICE)],
                semst).start()

    def stage_wait(w):
        @pl.when(w < NWIN - 1)
        def _():
            pltpu.make_async_copy(
                table_hbm.at[pl.ds(0, WSLICE)],
                shared.at[pl.ds(0, WSLICE)], semst).wait()

        @pl.when(w == NWIN - 1)
        def _():
            pltpu.make_async_copy(
                table_hbm.at[pl.ds(0, WLSLICE)],
                shared.at[pl.ds(0, WLSLICE)], semst).wait()

    stage_start(0)

    pltpu.sync_copy(x_hbm.at[pl.ds(base, NB)], xbuf.at[pl.ds(0, NB)])
    for k in range(NV):
        xbuf[NB, pl.ds(k * 16, 16)] = zeros16

    def bucket(j, tmpr, hb):
        for k in range(NV):
            tmpr[pl.ds(k * 16, 16)] = xbuf[j, pl.ds(k * 16, 16)]
        hb[pl.ds(0, 16)] = zeros16
        nzv = zeros16
        for k in range(NV):
            v = tmpr[pl.ds(k * 16, 16)]
            nzv = nzv + jnp.where(v != 0, ones16, zeros16)
            wl = lax.shift_right_logical(v, WBITS)
            cnt, last = plsc.scan_count(wl)
            tot = plsc.load_gather(hb, [wl])
            plsc.store_scatter(hb, [wl], tot + cnt, mask=last)
        histv = hb[pl.ds(0, 16)]
        excl = plsc.cumsum(histv) - histv
        hb[pl.ds(0, 16)] = excl
        nz = jnp.sum(nzv, axis=0)
        words = [jnp.int32(0)] * 3
        for w in range(1, NWIN):
            ow = jnp.max(jnp.where(iota16 == w, excl, zeros16))
            words[(w - 1) // 4] = words[(w - 1) // 4] | (
                ow << (8 * ((w - 1) % 4)))
        for i in range(3):
            smc[j * 4 + i] = words[i]
        smc[j * 4 + 3] = nz
        for k in range(NV):
            v = tmpr[pl.ds(k * 16, 16)]
            wl = lax.shift_right_logical(v, WBITS)
            loc = v & wmask
            cnt, last = plsc.scan_count(wl)
            bp = plsc.load_gather(hb, [wl])
            plsc.store_scatter(xbuf.at[j], [bp + cnt - 1], loc)
            plsc.store_scatter(hb, [wl], bp + cnt, mask=last)
        fz = jnp.zeros((16,), jnp.float32)
        for r in range(8):
            obuf[j, pl.ds(r * 16, 16)] = fz

    def bucket2(jj):
        bucket(jj * 2, tmp.at[0], hbuf.at[0])
        bucket(jj * 2 + 1, tmp.at[1], hbuf.at[1])

    pl.loop(0, NB // 2)(bucket2)

    def bstart(j, w):
        kk = jnp.maximum(w - 1, 0)
        word = smc[j * 4 + (kk >> 2)]
        b = (word >> (8 * (kk & 3))) & 255
        return jnp.where(w == 0, 0, b)

    def bend(j, w):
        kk = jnp.minimum(w, NWIN - 2)
        word = smc[j * 4 + (kk >> 2)]
        b = (word >> (8 * (kk & 3))) & 255
        return jnp.where(w == NWIN - 1, LPAD, b)

    def chunks(o, e):
        return (jnp.minimum(e - o, GCAP) + 15) >> 4

    def issue_seg(j, o, e, offv, gbuf, sem):
        def go(i):
            ivec = xbuf[j, pl.ds(o + i * 16, 16)] + offv
            pltpu.make_async_copy(
                shared.at[ivec], gbuf.at[pl.ds(i * 16, 16)], sem).start()
        pl.loop(0, chunks(o, e))(go)

    def drain_seg(o, e, gbuf, sem):
        def wt(i):
            pltpu.make_async_copy(
                shared.at[iota16], gbuf.at[pl.ds(0, 16)], sem).wait()
        pl.loop(0, chunks(o, e))(wt)

    def accum_seg(o, e, gbuf, acc0):
        def body(l, acc):
            new = []
            for k in range(NG):
                wv = plsc.bitcast(gbuf[l, pl.ds(k * 32, 32)], jnp.int32)
                ev = plsc.bitcast(wv << 16, jnp.float32)
                od = plsc.bitcast(wv & himask, jnp.float32)
                new.append(acc[2 * k] + ev)
                new.append(acc[2 * k + 1] + od)
            return tuple(new)
        return lax.fori_loop(0, jnp.minimum(e - o, GCAP), body, acc0)

    def issue(j, w, offv, gbuf, sem):
        issue_seg(j, bstart(j, w), bend(j, w), offv, gbuf, sem)

    def process(j, w, offv, gbuf, sem):
        o = bstart(j, w)
        e = bend(j, w)
        drain_seg(o, e, gbuf, sem)
        acc = tuple(obuf[j, pl.ds(r * 16, 16)] for r in range(8))
        acc = accum_seg(o, e, gbuf, acc)
        nseg = jnp.where(e - o > GCAP, 1, 0)

        def seg(s, acc):
            so = o + (s + 1) * GCAP
            issue_seg(j, so, e, offv, gbuf, sem)
            drain_seg(so, e, gbuf, sem)
            return accum_seg(so, e, gbuf, acc)

        acc = pl.loop(0, jnp.maximum(nseg, 0), init_carry=acc)(seg)
        for r in range(8):
            obuf[j, pl.ds(r * 16, 16)] = acc[r]

    def window(w):
        stage_wait(w)
        plsc.subcore_barrier()

        @pl.when(w + 1 < NWIN)
        def _():
            stage_start(w + 1)

        offv = jnp.full((16,), (w & 1) * WROWS, jnp.int32)
        issue(0, w, offv, gbufa, sema)

        def step(jj):
            for t, (gb, sm, go, gs) in enumerate(
                    ((gbufa, sema, gbufb, semb), (gbufb, semb, gbufa, sema))):
                j = jj * 2 + t

                @pl.when(j + 1 < NB)
                def _():
                    issue(j + 1, w, offv, go, gs)

                process(j, w, offv, gb, sm)

        pl.loop(0, NB // 2)(step)

    pl.loop(0, NWIN)(window)

    fone = jnp.ones((16,), jnp.float32)

    def finalize(j):
        nz = smc[j * 4 + 3]
        totv = jnp.full((16,), nz, jnp.int32).astype(jnp.float32)
        inv = fone / jnp.maximum(totv, fone)
        acc = tuple(obuf[j, pl.ds(r * 16, 16)] for r in range(8))
        orow = obuf.at[j]
        for k in range(NG):
            idx = iota16 * 2 + (k * 32)
            plsc.store_scatter(orow, [idx], acc[2 * k] * inv)
            plsc.store_scatter(orow, [idx + 1], acc[2 * k + 1] * inv)

    pl.loop(0, NB)(finalize)
    pltpu.sync_copy(obuf, out_hbm.at[pl.ds(base, NB)])


@jax.jit
def kernel(x, table):
    xpad = jnp.zeros((BATCH, LPAD), jnp.int32)
    xpad = xpad.at[:, :HIST_LEN].set(x.astype(jnp.int32))
    tb16 = table.astype(jnp.bfloat16)
    mesh = plsc.VectorSubcoreMesh(core_axis_name="c", subcore_axis_name="s")
    f = pl.kernel(
        _sc_kernel,
        out_type=jax.ShapeDtypeStruct((BATCH, EMBED_DIM), jnp.float32),
        mesh=mesh,
        compiler_params=pltpu.CompilerParams(
            use_tc_tiling_on_sc=False, needs_layout_passes=False),
        scratch_types=[
            pltpu.VMEM((NB + 1, LPAD), jnp.int32),
            pltpu.VMEM((2, LPAD), jnp.int32),
            pltpu.VMEM((2, 16), jnp.int32),
            pltpu.VMEM((GCAP, EMBED_DIM), jnp.bfloat16),
            pltpu.VMEM((GCAP, EMBED_DIM), jnp.bfloat16),
            pltpu.VMEM((NB, EMBED_DIM), jnp.float32),
            pltpu.VMEM_SHARED((2 * WROWS, EMBED_DIM), jnp.bfloat16),
            pltpu.SMEM((4 * NB,), jnp.int32),
            pltpu.SemaphoreType.DMA,
            pltpu.SemaphoreType.DMA,
            pltpu.SemaphoreType.DMA,
        ],
    )
    return f(xpad, tb16)


# speedup vs baseline: 1.0549x; 1.0549x over previous
import jax
import jax.numpy as jnp
from jax import lax
from jax.experimental import pallas as pl
from jax.experimental.pallas import tpu as pltpu
from jax.experimental.pallas import tpu_sc as plsc

VOCAB = 100000
EMBED_DIM = 128
BATCH = 4096
HIST_LEN = 200

NC = 2
NS = 16
NW = NC * NS
NB = BATCH // NW
LPAD = 208
NV = LPAD // 16
NG = EMBED_DIM // 32
WBITS = 14
WROWS = 1 << WBITS
NWIN = 7
WLAST = VOCAB - (NWIN - 1) * WROWS
WSLICE = WROWS // NS
WLSLICE = WLAST // NS
GCAP = 144


def _sc_kernel(x_hbm, table_hbm, out_hbm, xbuf, tmp, hbuf, gbufa, gbufb,
               obuf, shared, smc, sema, semb):
    wid = lax.axis_index("s") * NC + lax.axis_index("c")
    sid = lax.axis_index("s")
    base = wid * NB

    iota16 = lax.iota(jnp.int32, 16)
    ones16 = jnp.ones((16,), jnp.int32)
    zeros16 = jnp.zeros((16,), jnp.int32)
    himask = jnp.full((16,), jnp.int32(-65536))
    wmask = jnp.full((16,), jnp.int32(WROWS - 1))

    def stage(w):
        @pl.when(w < NWIN - 1)
        def _():
            pltpu.sync_copy(
                table_hbm.at[pl.ds(w * WROWS + sid * WSLICE, WSLICE)],
                shared.at[pl.ds(sid * WSLICE, WSLICE)])

        @pl.when(w == NWIN - 1)
        def _():
            pltpu.sync_copy(
                table_hbm.at[pl.ds((NWIN - 1) * WROWS + sid * WLSLICE,
                                   WLSLICE)],
                shared.at[pl.ds(sid * WLSLICE, WLSLICE)])

    pltpu.sync_copy(x_hbm.at[pl.ds(base, NB)], xbuf.at[pl.ds(0, NB)])
    for k in range(NV):
        xbuf[NB, pl.ds(k * 16, 16)] = zeros16

    def bucket(j, tmpr, hb):
        for k in range(NV):
            tmpr[pl.ds(k * 16, 16)] = xbuf[j, pl.ds(k * 16, 16)]
        hb[pl.ds(0, 16)] = zeros16
        nzv = zeros16
        for k in range(NV):
            v = tmpr[pl.ds(k * 16, 16)]
            nzv = nzv + jnp.where(v != 0, ones16, zeros16)
            wl = lax.shift_right_logical(v, WBITS)
            cnt, last = plsc.scan_count(wl)
            tot = plsc.load_gather(hb, [wl])
            plsc.store_scatter(hb, [wl], tot + cnt, mask=last)
        histv = hb[pl.ds(0, 16)]
        excl = plsc.cumsum(histv) - histv
        hb[pl.ds(0, 16)] = excl
        nz = jnp.sum(nzv, axis=0)
        words = [jnp.int32(0)] * 3
        for w in range(1, NWIN):
            ow = jnp.max(jnp.where(iota16 == w, excl, zeros16))
            words[(w - 1) // 4] = words[(w - 1) // 4] | (
                ow << (8 * ((w - 1) % 4)))
        for i in range(3):
            smc[j * 4 + i] = words[i]
        smc[j * 4 + 3] = nz
        for k in range(NV):
            v = tmpr[pl.ds(k * 16, 16)]
            wl = lax.shift_right_logical(v, WBITS)
            loc = v & wmask
            cnt, last = plsc.scan_count(wl)
            bp = plsc.load_gather(hb, [wl])
            plsc.store_scatter(xbuf.at[j], [bp + cnt - 1], loc)
            plsc.store_scatter(hb, [wl], bp + cnt, mask=last)
        fz = jnp.zeros((16,), jnp.float32)
        for r in range(8):
            obuf[j, pl.ds(r * 16, 16)] = fz

    def bucket2(jj):
        bucket(jj * 2, tmp.at[0], hbuf.at[0])
        bucket(jj * 2 + 1, tmp.at[1], hbuf.at[1])

    pl.loop(0, NB // 2)(bucket2)

    def bstart(j, w):
        kk = jnp.maximum(w - 1, 0)
        word = smc[j * 4 + (kk >> 2)]
        b = (word >> (8 * (kk & 3))) & 255
        return jnp.where(w == 0, 0, b)

    def bend(j, w):
        kk = jnp.minimum(w, NWIN - 2)
        word = smc[j * 4 + (kk >> 2)]
        b = (word >> (8 * (kk & 3))) & 255
        return jnp.where(w == NWIN - 1, LPAD, b)

    def chunks(o, e):
        return (jnp.minimum(e - o, GCAP) + 15) >> 4

    def issue_seg(j, o, e, offv, gbuf, sem):
        def go(i):
            ivec = xbuf[j, pl.ds(o + i * 16, 16)] + offv
            pltpu.make_async_copy(
                shared.at[ivec], gbuf.at[pl.ds(i * 16, 16)], sem).start()
        pl.loop(0, chunks(o, e))(go)

    def drain_seg(o, e, gbuf, sem):
        def wt(i):
            pltpu.make_async_copy(
                shared.at[iota16], gbuf.at[pl.ds(0, 16)], sem).wait()
        pl.loop(0, chunks(o, e))(wt)

    def accum_seg(o, e, gbuf, acc0):
        def body(l, acc):
            new = []
            for k in range(NG):
                wv = plsc.bitcast(gbuf[l, pl.ds(k * 32, 32)], jnp.int32)
                ev = plsc.bitcast(wv << 16, jnp.float32)
                od = plsc.bitcast(wv & himask, jnp.float32)
                new.append(acc[2 * k] + ev)
                new.append(acc[2 * k + 1] + od)
            return tuple(new)
        return lax.fori_loop(0, jnp.minimum(e - o, GCAP), body, acc0)

    def issue(j, w, offv, gbuf, sem):
        issue_seg(j, bstart(j, w), bend(j, w), offv, gbuf, sem)

    def process(j, w, offv, gbuf, sem):
        o = bstart(j, w)
        e = bend(j, w)
        drain_seg(o, e, gbuf, sem)
        acc = tuple(obuf[j, pl.ds(r * 16, 16)] for r in range(8))
        acc = accum_seg(o, e, gbuf, acc)
        nseg = jnp.where(e - o > GCAP, 1, 0)

        def seg(s, acc):
            so = o + (s + 1) * GCAP
            issue_seg(j, so, e, offv, gbuf, sem)
            drain_seg(so, e, gbuf, sem)
            return accum_seg(so, e, gbuf, acc)

        acc = pl.loop(0, jnp.maximum(nseg, 0), init_carry=acc)(seg)
        for r in range(8):
            obuf[j, pl.ds(r * 16, 16)] = acc[r]

    def window(w):
        plsc.subcore_barrier()
        stage(w)
        plsc.subcore_barrier()

        offv = zeros16
        issue(0, w, offv, gbufa, sema)

        def step(jj):
            for t, (gb, sm, go, gs) in enumerate(
                    ((gbufa, sema, gbufb, semb), (gbufb, semb, gbufa, sema))):
                j = jj * 2 + t

                @pl.when(j + 1 < NB)
                def _():
                    issue(j + 1, w, offv, go, gs)

                process(j, w, offv, gb, sm)

        pl.loop(0, NB // 2)(step)

    pl.loop(0, NWIN)(window)

    fone = jnp.ones((16,), jnp.float32)

    def finalize(j):
        nz = smc[j * 4 + 3]
        totv = jnp.full((16,), nz, jnp.int32).astype(jnp.float32)
        inv = fone / jnp.maximum(totv, fone)
        acc = tuple(obuf[j, pl.ds(r * 16, 16)] for r in range(8))
        orow = obuf.at[j]
        for k in range(NG):
            idx = iota16 * 2 + (k * 32)
            plsc.store_scatter(orow, [idx], acc[2 * k] * inv)
            plsc.store_scatter(orow, [idx + 1], acc[2 * k + 1] * inv)

    pl.loop(0, NB)(finalize)
    pltpu.sync_copy(obuf, out_hbm.at[pl.ds(base, NB)])


@jax.jit
def kernel(x, table):
    xpad = jnp.zeros((BATCH, LPAD), jnp.int32)
    xpad = xpad.at[:, :HIST_LEN].set(x.astype(jnp.int32))
    tb16 = table.astype(jnp.bfloat16)
    mesh = plsc.VectorSubcoreMesh(core_axis_name="c", subcore_axis_name="s")
    f = pl.kernel(
        _sc_kernel,
        out_type=jax.ShapeDtypeStruct((BATCH, EMBED_DIM), jnp.float32),
        mesh=mesh,
        compiler_params=pltpu.CompilerParams(
            use_tc_tiling_on_sc=False, needs_layout_passes=False),
        scratch_types=[
            pltpu.VMEM((NB + 1, LPAD), jnp.int32),
            pltpu.VMEM((2, LPAD), jnp.int32),
            pltpu.VMEM((2, 16), jnp.int32),
            pltpu.VMEM((GCAP, EMBED_DIM), jnp.bfloat16),
            pltpu.VMEM((GCAP, EMBED_DIM), jnp.bfloat16),
            pltpu.VMEM((NB, EMBED_DIM), jnp.float32),
            pltpu.VMEM_SHARED((WROWS, EMBED_DIM), jnp.bfloat16),
            pltpu.SMEM((4 * NB,), jnp.int32),
            pltpu.SemaphoreType.DMA,
            pltpu.SemaphoreType.DMA,
        ],
    )
    return f(xpad, tb16)
